# baseline (speedup 1.0000x reference)
import functools

import jax
import jax.numpy as jnp
from jax import lax
from jax.experimental import pallas as pl
from jax.experimental.pallas import tpu as pltpu
from jax.experimental.pallas import tpu_sc as plsc

NC = 2
NS = 16
NW = NC * NS

COORDS_RANGE_OVER_NORM = 15.0 / 100.0


def _ab_body(h_ref, wa_ref, wb_ref, a_ref, b_ref):
    hb = h_ref[...].astype(jnp.bfloat16)
    dn = (((1,), (1,)), ((), ()))
    a_ref[...] = lax.dot_general(
        hb, wa_ref[...], dn, preferred_element_type=jnp.float32)
    b_ref[...] = lax.dot_general(
        hb, wb_ref[...], dn, preferred_element_type=jnp.float32)


def _node_ab(h, w1a, w1b):
    n, hdim = h.shape
    return pl.pallas_call(
        _ab_body,
        out_shape=(
            jax.ShapeDtypeStruct((n, hdim), jnp.float32),
            jax.ShapeDtypeStruct((n, hdim), jnp.float32),
        ),
    )(h, w1a, w1b)


IW = 40
TPC = 5
CG = IW * TPC
NSLOT = 2


def _gather_pair(a_t, b_t, row1d, col1d, e):
    epw = e // NW
    nchunks = epw // CG
    mesh = plsc.VectorSubcoreMesh(core_axis_name="c", subcore_axis_name="s")
    hdim = a_t.shape[1]

    @functools.partial(
        pl.kernel,
        out_type=(
            jax.ShapeDtypeStruct((e, hdim), jnp.float32),
            jax.ShapeDtypeStruct((e, hdim), jnp.float32),
        ),
        mesh=mesh,
        scratch_types=[
            pltpu.VMEM((NSLOT * CG,), jnp.int32),
            pltpu.VMEM((NSLOT * CG,), jnp.int32),
            pltpu.VMEM((NSLOT * CG, hdim), jnp.float32),
            pltpu.VMEM((NSLOT * CG, hdim), jnp.float32),
            pltpu.SemaphoreType.DMA,
            pltpu.SemaphoreType.DMA,
        ],
    )
    def k(a_hbm, b_hbm, row_hbm, col_hbm, ga_hbm, gb_hbm,
          rowv, colv, bufa, bufb, semg, semw):
        wid = lax.axis_index("c") * NS + lax.axis_index("s")
        base0 = wid * epw

        wb = {}
        for c in range(nchunks):
            slot = c % NSLOT
            so = slot * CG
            if c >= NSLOT:
                for d in wb.pop(c - NSLOT):
                    d.wait()
            base = base0 + c * CG
            pltpu.sync_copy(row_hbm.at[pl.ds(base, CG)],
                            rowv.at[pl.ds(so, CG)])
            pltpu.sync_copy(col_hbm.at[pl.ds(base, CG)],
                            colv.at[pl.ds(so, CG)])
            gd = []
            for j in range(TPC):
                gd.append(pltpu.async_copy(
                    a_hbm.at[rowv.at[pl.ds(so + j * IW, IW)]],
                    bufa.at[pl.ds(so + j * IW, IW)], semg))
                gd.append(pltpu.async_copy(
                    b_hbm.at[colv.at[pl.ds(so + j * IW, IW)]],
                    bufb.at[pl.ds(so + j * IW, IW)], semg))
            for d in gd:
                d.wait()
            wb[c] = [
                pltpu.async_copy(bufa.at[pl.ds(so, CG)],
                                 ga_hbm.at[pl.ds(base, CG)], semw),
                pltpu.async_copy(bufb.at[pl.ds(so, CG)],
                                 gb_hbm.at[pl.ds(base, CG)], semw),
            ]
        for c in sorted(wb):
            for d in wb[c]:
                d.wait()

    return k(a_t, b_t, row1d, col1d)


BE = 512


def _mlp_body(ga_ref, gb_ref, ea_ref, w1c_ref, b1_ref, w2_ref,
              b2_ref, w3_ref, t_ref):
    dn = (((1,), (1,)), ((), ()))
    pre = ga_ref[...] + gb_ref[...]
    pre = pre + lax.dot_general(
        ea_ref[...].astype(jnp.bfloat16), w1c_ref[...], dn,
        preferred_element_type=jnp.float32)
    pre = (pre + b1_ref[...]).astype(jnp.bfloat16)
    x = pre * jax.nn.sigmoid(pre)
    pre2 = (lax.dot_general(
        x, w2_ref[...], dn, preferred_element_type=jnp.float32)
        + b2_ref[...]).astype(jnp.bfloat16)
    y = pre2 * jax.nn.sigmoid(pre2)
    s = lax.dot_general(
        w3_ref[...], y, dn, preferred_element_type=jnp.float32)
    t_ref[...] = (jnp.tanh(s) * COORDS_RANGE_OVER_NORM)[0]


def _edge_mlp(ga, gb, ea, w1c, b1r, w2, b2r, w3):
    e, hdim = ga.shape
    de = ea.shape[1]
    grid = (e // BE,)

    return pl.pallas_call(
        _mlp_body,
        grid=grid,
        in_specs=[
            pl.BlockSpec((BE, hdim), lambda i: (i, 0)),
            pl.BlockSpec((BE, hdim), lambda i: (i, 0)),
            pl.BlockSpec((BE, de), lambda i: (i, 0)),
            pl.BlockSpec((hdim, de), lambda i: (0, 0)),
            pl.BlockSpec((1, hdim), lambda i: (0, 0)),
            pl.BlockSpec((hdim, hdim), lambda i: (0, 0)),
            pl.BlockSpec((1, hdim), lambda i: (0, 0)),
            pl.BlockSpec((1, hdim), lambda i: (0, 0)),
        ],
        out_specs=pl.BlockSpec((BE,), lambda i: (i,)),
        out_shape=jax.ShapeDtypeStruct((e,), jnp.float32),
    )(ga, gb, ea, w1c, b1r, w2, b2r, w3)


ACC = 32768
C4 = 2000


def _segment_scatter(t_all, cdflat, row1d, zeros1d, e):
    epw = e // NW
    mesh = plsc.VectorSubcoreMesh(core_axis_name="c", subcore_axis_name="s")
    sl = ACC // NS

    @functools.partial(
        pl.kernel,
        out_type=jax.ShapeDtypeStruct((NC, ACC // 128, 128), jnp.float32),
        mesh=mesh,
        scratch_types=[
            pltpu.VMEM((C4,), jnp.int32),
            pltpu.VMEM((C4,), jnp.float32),
            pltpu.VMEM((3 * C4,), jnp.float32),
            pltpu.VMEM((ACC,), jnp.float32),
            pltpu.VMEM((NS, sl), jnp.float32),
            pltpu.VMEM((sl // 128, 128), jnp.float32),
            pltpu.VMEM_SHARED((NS, ACC), jnp.float32),
        ],
        compiler_params=pltpu.CompilerParams(needs_layout_passes=False),
    )
    def k(t_hbm, cd_hbm, row_hbm, zero_hbm, out_hbm,
          rowv, tv, cdv, accl, buf2, res, stage):
        cid = lax.axis_index("c")
        sid = lax.axis_index("s")
        wid = cid * NS + sid

        pltpu.sync_copy(zero_hbm, accl)
        i3 = lax.iota(jnp.int32, 16) * 3

        def chunk(i, carry):
            base = wid * epw + i * C4
            pltpu.sync_copy(row_hbm.at[pl.ds(base, C4)], rowv)
            pltpu.sync_copy(t_hbm.at[pl.ds(base, C4)], tv)
            pltpu.sync_copy(cd_hbm.at[pl.ds(3 * base, 3 * C4)], cdv)

            def grp(g, c2):
                rv = rowv[pl.ds(g * 16, 16)]
                th = tv[pl.ds(g * 16, 16)]
                f0 = rv * 3
                c0 = i3 + g * 48
                for d in range(3):
                    cd_d = plsc.load_gather(cdv, [c0 + d])
                    plsc.addupdate_scatter(accl, [f0 + d], cd_d * th)
                return c2

            lax.fori_loop(0, C4 // 16, grp, 0)
            return carry

        lax.fori_loop(0, epw // C4, chunk, 0)

        pltpu.sync_copy(accl, stage.at[sid])
        plsc.subcore_barrier()
        pltpu.sync_copy(stage.at[:, pl.ds(sid * sl, sl)], buf2)

        def colsum_rows(jr, carry):
            for jc in range(8):
                j = jr * 8 + jc
                acc16 = buf2[0, pl.ds(j * 16, 16)]
                for r in range(1, NS):
                    acc16 = acc16 + buf2[r, pl.ds(j * 16, 16)]
                res[jr, pl.ds(jc * 16, 16)] = acc16
            return carry

        lax.fori_loop(0, sl // 128, colsum_rows, 0)
        pltpu.sync_copy(res, out_hbm.at[cid, pl.ds(sid * (sl // 128),
                                                   sl // 128)])

    return k(t_all, cdflat, row1d, zeros1d)


def kernel(h, coord, edge_index, coord_diff, edge_attr, W1, b1, W2, b2, W3):
    n, hdim = h.shape
    e = edge_index.shape[1]

    w1a = W1[:, :hdim].astype(jnp.bfloat16)
    w1b = W1[:, hdim:2 * hdim].astype(jnp.bfloat16)
    w1c = W1[:, 2 * hdim:].astype(jnp.bfloat16)

    a_t, b_t = _node_ab(h, w1a, w1b)

    row = edge_index[0]
    col = edge_index[1]
    ga, gb = _gather_pair(a_t, b_t, row, col, e)

    t_all = _edge_mlp(
        ga, gb, edge_attr, w1c,
        b1.reshape(1, -1), W2.astype(jnp.bfloat16), b2.reshape(1, -1),
        W3.astype(jnp.bfloat16))

    zeros1d = jnp.zeros((ACC,), dtype=jnp.float32)
    partials = _segment_scatter(t_all, coord_diff.reshape(3 * e), row,
                                zeros1d, e)

    agg = (partials[0] + partials[1]).reshape(-1)[:3 * n].reshape(n, 3)
    return coord + agg


# speedup vs baseline: 1.3887x; 1.3887x over previous
import functools

import jax
import jax.numpy as jnp
from jax import lax
from jax.experimental import pallas as pl
from jax.experimental.pallas import tpu as pltpu
from jax.experimental.pallas import tpu_sc as plsc

NC = 2
NS = 16
NW = NC * NS

COORDS_RANGE_OVER_NORM = 15.0 / 100.0


def _ab_body(h_ref, wa_ref, wb_ref, a_ref, b_ref):
    hb = h_ref[...].astype(jnp.bfloat16)
    dn = (((1,), (1,)), ((), ()))
    a_ref[...] = lax.dot_general(
        hb, wa_ref[...], dn, preferred_element_type=jnp.float32)
    b_ref[...] = lax.dot_general(
        hb, wb_ref[...], dn, preferred_element_type=jnp.float32)


def _node_ab(h, w1a, w1b):
    n, hdim = h.shape
    return pl.pallas_call(
        _ab_body,
        out_shape=(
            jax.ShapeDtypeStruct((n, hdim), jnp.float32),
            jax.ShapeDtypeStruct((n, hdim), jnp.float32),
        ),
    )(h, w1a, w1b)


IW = 80
TPC = 5
CG = IW * TPC
NSLOT = 2


def _gather_sum(a_t, b_t, row1d, col1d, e):
    epw = e // NW
    nchunks = epw // CG
    mesh = plsc.VectorSubcoreMesh(core_axis_name="c", subcore_axis_name="s")
    hdim = a_t.shape[1]

    @functools.partial(
        pl.kernel,
        out_type=jax.ShapeDtypeStruct((e, hdim), jnp.float32),
        mesh=mesh,
        scratch_types=[
            pltpu.VMEM((NSLOT * CG,), jnp.int32),
            pltpu.VMEM((NSLOT * CG,), jnp.int32),
            pltpu.VMEM((NSLOT * CG, hdim), jnp.float32),
            pltpu.SemaphoreType.DMA,
            pltpu.SemaphoreType.DMA,
        ],
    )
    def k(a_hbm, b_hbm, row_hbm, col_hbm, g_hbm, rowv, colv, buf, semg, semw):
        wid = lax.axis_index("c") * NS + lax.axis_index("s")
        base0 = wid * epw

        wb = {}
        for c in range(nchunks):
            slot = c % NSLOT
            so = slot * CG
            if c >= NSLOT:
                wb.pop(c - NSLOT).wait()
            base = base0 + c * CG
            pltpu.sync_copy(row_hbm.at[pl.ds(base, CG)],
                            rowv.at[pl.ds(so, CG)])
            pltpu.sync_copy(col_hbm.at[pl.ds(base, CG)],
                            colv.at[pl.ds(so, CG)])
            gd = []
            for j in range(TPC):
                gd.append(pltpu.async_copy(
                    a_hbm.at[rowv.at[pl.ds(so + j * IW, IW)]],
                    buf.at[pl.ds(so + j * IW, IW)], semg))
            for d in gd:
                d.wait()
            gd = []
            for j in range(TPC):
                gd.append(pltpu.async_copy(
                    b_hbm.at[colv.at[pl.ds(so + j * IW, IW)]],
                    buf.at[pl.ds(so + j * IW, IW)], semg, add=True))
            for d in gd:
                d.wait()
            wb[c] = pltpu.async_copy(buf.at[pl.ds(so, CG)],
                                     g_hbm.at[pl.ds(base, CG)], semw)
        for c in sorted(wb):
            wb[c].wait()

    return k(a_t, b_t, row1d, col1d)


BE = 512


def _mlp_body(g_ref, ea_ref, w1c_ref, b1_ref, w2_ref, b2_ref, w3_ref, t_ref):
    dn = (((1,), (1,)), ((), ()))
    pre = g_ref[...] + lax.dot_general(
        ea_ref[...].astype(jnp.bfloat16), w1c_ref[...],
        (((0,), (1,)), ((), ())), preferred_element_type=jnp.float32)
    pre = (pre + b1_ref[...]).astype(jnp.bfloat16)
    x = pre * jax.nn.sigmoid(pre)
    pre2 = (lax.dot_general(
        x, w2_ref[...], dn, preferred_element_type=jnp.float32)
        + b2_ref[...]).astype(jnp.bfloat16)
    y = pre2 * jax.nn.sigmoid(pre2)
    s = lax.dot_general(
        w3_ref[...], y, dn, preferred_element_type=jnp.float32)
    t_ref[...] = (jnp.tanh(s) * COORDS_RANGE_OVER_NORM)[0]


def _edge_mlp(g, ea_t, w1c, b1r, w2, b2r, w3):
    e, hdim = g.shape
    de = ea_t.shape[0]
    grid = (e // BE,)

    return pl.pallas_call(
        _mlp_body,
        grid=grid,
        in_specs=[
            pl.BlockSpec((BE, hdim), lambda i: (i, 0)),
            pl.BlockSpec((de, BE), lambda i: (0, i)),
            pl.BlockSpec((hdim, de), lambda i: (0, 0)),
            pl.BlockSpec((1, hdim), lambda i: (0, 0)),
            pl.BlockSpec((hdim, hdim), lambda i: (0, 0)),
            pl.BlockSpec((1, hdim), lambda i: (0, 0)),
            pl.BlockSpec((1, hdim), lambda i: (0, 0)),
        ],
        out_specs=pl.BlockSpec((BE,), lambda i: (i,)),
        out_shape=jax.ShapeDtypeStruct((e,), jnp.float32),
    )(g, ea_t, w1c, b1r, w2, b2r, w3)


ACC = 32768
C4 = 2000


def _segment_scatter(t_all, cd0, cd1, cd2, row1d, zeros1d, e):
    epw = e // NW
    mesh = plsc.VectorSubcoreMesh(core_axis_name="c", subcore_axis_name="s")
    sl = ACC // NS

    @functools.partial(
        pl.kernel,
        out_type=jax.ShapeDtypeStruct((NC, ACC // 128, 128), jnp.float32),
        mesh=mesh,
        scratch_types=[
            pltpu.VMEM((C4,), jnp.int32),
            pltpu.VMEM((C4,), jnp.float32),
            pltpu.VMEM((C4,), jnp.float32),
            pltpu.VMEM((C4,), jnp.float32),
            pltpu.VMEM((C4,), jnp.float32),
            pltpu.VMEM((ACC,), jnp.float32),
            pltpu.VMEM((NS, sl), jnp.float32),
            pltpu.VMEM((sl // 128, 128), jnp.float32),
            pltpu.VMEM_SHARED((NS, ACC), jnp.float32),
        ],
        compiler_params=pltpu.CompilerParams(needs_layout_passes=False),
    )
    def k(t_hbm, cd0_hbm, cd1_hbm, cd2_hbm, row_hbm, zero_hbm, out_hbm,
          rowv, tv, c0v, c1v, c2v, accl, buf2, res, stage):
        cid = lax.axis_index("c")
        sid = lax.axis_index("s")
        wid = cid * NS + sid

        pltpu.sync_copy(zero_hbm, accl)

        def chunk(i, carry):
            base = wid * epw + i * C4
            pltpu.sync_copy(row_hbm.at[pl.ds(base, C4)], rowv)
            pltpu.sync_copy(t_hbm.at[pl.ds(base, C4)], tv)
            pltpu.sync_copy(cd0_hbm.at[pl.ds(base, C4)], c0v)
            pltpu.sync_copy(cd1_hbm.at[pl.ds(base, C4)], c1v)
            pltpu.sync_copy(cd2_hbm.at[pl.ds(base, C4)], c2v)

            def grp(g, c2):
                rv = rowv[pl.ds(g * 16, 16)]
                th = tv[pl.ds(g * 16, 16)]
                f0 = rv * 3
                for d, cdv in enumerate((c0v, c1v, c2v)):
                    cd_d = cdv[pl.ds(g * 16, 16)]
                    plsc.addupdate_scatter(accl, [f0 + d], cd_d * th)
                return c2

            lax.fori_loop(0, C4 // 16, grp, 0)
            return carry

        lax.fori_loop(0, epw // C4, chunk, 0)

        pltpu.sync_copy(accl, stage.at[sid])
        plsc.subcore_barrier()
        pltpu.sync_copy(stage.at[:, pl.ds(sid * sl, sl)], buf2)

        def colsum_rows(jr, carry):
            for jc in range(8):
                j = jr * 8 + jc
                acc16 = buf2[0, pl.ds(j * 16, 16)]
                for r in range(1, NS):
                    acc16 = acc16 + buf2[r, pl.ds(j * 16, 16)]
                res[jr, pl.ds(jc * 16, 16)] = acc16
            return carry

        lax.fori_loop(0, sl // 128, colsum_rows, 0)
        pltpu.sync_copy(res, out_hbm.at[cid, pl.ds(sid * (sl // 128),
                                                   sl // 128)])

    return k(t_all, cd0, cd1, cd2, row1d, zeros1d)


def kernel(h, coord, edge_index, coord_diff, edge_attr, W1, b1, W2, b2, W3):
    n, hdim = h.shape
    e = edge_index.shape[1]

    w1a = W1[:, :hdim].astype(jnp.bfloat16)
    w1b = W1[:, hdim:2 * hdim].astype(jnp.bfloat16)
    w1c = W1[:, 2 * hdim:].astype(jnp.bfloat16)

    a_t, b_t = _node_ab(h, w1a, w1b)

    row = edge_index[0]
    col = edge_index[1]
    g = _gather_sum(a_t, b_t, row, col, e)

    t_all = _edge_mlp(
        g, edge_attr.T, w1c,
        b1.reshape(1, -1), W2.astype(jnp.bfloat16), b2.reshape(1, -1),
        W3.astype(jnp.bfloat16))

    zeros1d = jnp.zeros((ACC,), dtype=jnp.float32)
    partials = _segment_scatter(t_all, coord_diff[:, 0], coord_diff[:, 1],
                                coord_diff[:, 2], row, zeros1d, e)

    agg = (partials[0] + partials[1]).reshape(-1)[:3 * n].reshape(n, 3)
    return coord + agg


# speedup vs baseline: 2.2274x; 1.6039x over previous
import functools

import jax
import jax.numpy as jnp
from jax import lax
from jax.experimental import pallas as pl
from jax.experimental.pallas import tpu as pltpu
from jax.experimental.pallas import tpu_sc as plsc

NC = 2
NS = 16
NW = NC * NS

COORDS_RANGE_OVER_NORM = 15.0 / 100.0


def _ab_body(h_ref, wa_ref, wb_ref, a_ref, b_ref):
    hb = h_ref[...].astype(jnp.bfloat16)
    dn = (((1,), (1,)), ((), ()))
    a_ref[...] = lax.dot_general(
        hb, wa_ref[...], dn, preferred_element_type=jnp.float32)
    b_ref[...] = lax.dot_general(
        hb, wb_ref[...], dn, preferred_element_type=jnp.float32)


def _node_ab(h, w1a, w1b):
    n, hdim = h.shape
    return pl.pallas_call(
        _ab_body,
        out_shape=(
            jax.ShapeDtypeStruct((n, hdim), jnp.float32),
            jax.ShapeDtypeStruct((n, hdim), jnp.float32),
        ),
    )(h, w1a, w1b)


IW = 80
TPC = 5
CG = IW * TPC
NSLOT = 2


def _gather_sum(a_t, b_t, row1d, col1d, e):
    epw = e // NW
    nchunks = epw // CG
    mesh = plsc.VectorSubcoreMesh(core_axis_name="c", subcore_axis_name="s")
    hdim = a_t.shape[1]

    @functools.partial(
        pl.kernel,
        out_type=jax.ShapeDtypeStruct((EPAD, hdim), jnp.float32),
        mesh=mesh,
        scratch_types=[
            pltpu.VMEM((NSLOT * CG,), jnp.int32),
            pltpu.VMEM((NSLOT * CG,), jnp.int32),
            pltpu.VMEM((NSLOT * CG, hdim), jnp.float32),
            pltpu.SemaphoreType.DMA,
            pltpu.SemaphoreType.DMA,
        ],
    )
    def k(a_hbm, b_hbm, row_hbm, col_hbm, g_hbm, rowv, colv, buf, semg, semw):
        wid = lax.axis_index("c") * NS + lax.axis_index("s")
        base0 = wid * epw

        wb = {}
        for c in range(nchunks):
            slot = c % NSLOT
            so = slot * CG
            if c >= NSLOT:
                wb.pop(c - NSLOT).wait()
            base = base0 + c * CG
            pltpu.sync_copy(row_hbm.at[pl.ds(base, CG)],
                            rowv.at[pl.ds(so, CG)])
            pltpu.sync_copy(col_hbm.at[pl.ds(base, CG)],
                            colv.at[pl.ds(so, CG)])
            gd = []
            for j in range(TPC):
                gd.append(pltpu.async_copy(
                    a_hbm.at[rowv.at[pl.ds(so + j * IW, IW)]],
                    buf.at[pl.ds(so + j * IW, IW)], semg))
            for d in gd:
                d.wait()
            gd = []
            for j in range(TPC):
                gd.append(pltpu.async_copy(
                    b_hbm.at[colv.at[pl.ds(so + j * IW, IW)]],
                    buf.at[pl.ds(so + j * IW, IW)], semg, add=True))
            for d in gd:
                d.wait()
            wb[c] = pltpu.async_copy(buf.at[pl.ds(so, CG)],
                                     g_hbm.at[pl.ds(base, CG)], semw)
        for c in sorted(wb):
            wb[c].wait()

    return k(a_t, b_t, row1d, col1d)


BE = 2048
EPAD = 327680


def _mlp_body(g_ref, ea_ref, w1c_ref, b1_ref, w2_ref, b2_ref, w3_ref, t_ref):
    dn = (((1,), (1,)), ((), ()))
    pre = g_ref[...] + lax.dot_general(
        ea_ref[...].astype(jnp.bfloat16), w1c_ref[...],
        (((0,), (1,)), ((), ())), preferred_element_type=jnp.float32)
    pre = (pre + b1_ref[...]).astype(jnp.bfloat16)
    x = pre * jax.nn.sigmoid(pre)
    pre2 = (lax.dot_general(
        x, w2_ref[...], dn, preferred_element_type=jnp.float32)
        + b2_ref[...]).astype(jnp.bfloat16)
    y = pre2 * jax.nn.sigmoid(pre2)
    s = lax.dot_general(
        w3_ref[...], y, dn, preferred_element_type=jnp.float32)
    t_ref[...] = (jnp.tanh(s) * COORDS_RANGE_OVER_NORM)[0]


def _edge_mlp(g, ea_t, w1c, b1r, w2, b2r, w3):
    epad, hdim = g.shape
    de = ea_t.shape[0]
    grid = (epad // BE,)

    return pl.pallas_call(
        _mlp_body,
        grid=grid,
        in_specs=[
            pl.BlockSpec((BE, hdim), lambda i: (i, 0)),
            pl.BlockSpec((de, BE), lambda i: (0, i)),
            pl.BlockSpec((hdim, de), lambda i: (0, 0)),
            pl.BlockSpec((1, hdim), lambda i: (0, 0)),
            pl.BlockSpec((hdim, hdim), lambda i: (0, 0)),
            pl.BlockSpec((1, hdim), lambda i: (0, 0)),
            pl.BlockSpec((1, hdim), lambda i: (0, 0)),
        ],
        out_specs=pl.BlockSpec((BE,), lambda i: (i,)),
        out_shape=jax.ShapeDtypeStruct((epad,), jnp.float32),
    )(g, ea_t, w1c, b1r, w2, b2r, w3)


ACC = 32768
C4 = 2000


def _segment_scatter(t_all, cd0, cd1, cd2, row1d, zeros1d, e):
    epw = e // NW
    mesh = plsc.VectorSubcoreMesh(core_axis_name="c", subcore_axis_name="s")
    sl = ACC // NS

    @functools.partial(
        pl.kernel,
        out_type=jax.ShapeDtypeStruct((NC, ACC // 128, 128), jnp.float32),
        mesh=mesh,
        scratch_types=[
            pltpu.VMEM((C4,), jnp.int32),
            pltpu.VMEM((C4,), jnp.float32),
            pltpu.VMEM((C4,), jnp.float32),
            pltpu.VMEM((C4,), jnp.float32),
            pltpu.VMEM((C4,), jnp.float32),
            pltpu.VMEM((ACC,), jnp.float32),
            pltpu.VMEM((NS, sl), jnp.float32),
            pltpu.VMEM((sl // 128, 128), jnp.float32),
            pltpu.VMEM_SHARED((NS, ACC), jnp.float32),
        ],
        compiler_params=pltpu.CompilerParams(needs_layout_passes=False),
    )
    def k(t_hbm, cd0_hbm, cd1_hbm, cd2_hbm, row_hbm, zero_hbm, out_hbm,
          rowv, tv, c0v, c1v, c2v, accl, buf2, res, stage):
        cid = lax.axis_index("c")
        sid = lax.axis_index("s")
        wid = cid * NS + sid

        pltpu.sync_copy(zero_hbm, accl)

        def chunk(i, carry):
            base = wid * epw + i * C4
            pltpu.sync_copy(row_hbm.at[pl.ds(base, C4)], rowv)
            pltpu.sync_copy(t_hbm.at[pl.ds(base, C4)], tv)
            pltpu.sync_copy(cd0_hbm.at[pl.ds(base, C4)], c0v)
            pltpu.sync_copy(cd1_hbm.at[pl.ds(base, C4)], c1v)
            pltpu.sync_copy(cd2_hbm.at[pl.ds(base, C4)], c2v)

            def grp(g, c2):
                rv = rowv[pl.ds(g * 16, 16)]
                th = tv[pl.ds(g * 16, 16)]
                f0 = rv * 3
                for d, cdv in enumerate((c0v, c1v, c2v)):
                    cd_d = cdv[pl.ds(g * 16, 16)]
                    plsc.addupdate_scatter(accl, [f0 + d], cd_d * th)
                return c2

            lax.fori_loop(0, C4 // 16, grp, 0)
            return carry

        lax.fori_loop(0, epw // C4, chunk, 0)

        pltpu.sync_copy(accl, stage.at[sid])
        plsc.subcore_barrier()
        pltpu.sync_copy(stage.at[:, pl.ds(sid * sl, sl)], buf2)

        def colsum_rows(jr, carry):
            for jc in range(8):
                j = jr * 8 + jc
                acc16 = buf2[0, pl.ds(j * 16, 16)]
                for r in range(1, NS):
                    acc16 = acc16 + buf2[r, pl.ds(j * 16, 16)]
                res[jr, pl.ds(jc * 16, 16)] = acc16
            return carry

        lax.fori_loop(0, sl // 128, colsum_rows, 0)
        pltpu.sync_copy(res, out_hbm.at[cid, pl.ds(sid * (sl // 128),
                                                   sl // 128)])

    return k(t_all, cd0, cd1, cd2, row1d, zeros1d)


def kernel(h, coord, edge_index, coord_diff, edge_attr, W1, b1, W2, b2, W3):
    n, hdim = h.shape
    e = edge_index.shape[1]

    w1a = W1[:, :hdim].astype(jnp.bfloat16)
    w1b = W1[:, hdim:2 * hdim].astype(jnp.bfloat16)
    w1c = W1[:, 2 * hdim:].astype(jnp.bfloat16)

    a_t, b_t = _node_ab(h, w1a, w1b)

    row = edge_index[0]
    col = edge_index[1]
    g = _gather_sum(a_t, b_t, row, col, e)

    ea_t = jnp.pad(edge_attr, ((0, EPAD - e), (0, 0))).T
    t_all = _edge_mlp(
        g, ea_t, w1c,
        b1.reshape(1, -1), W2.astype(jnp.bfloat16), b2.reshape(1, -1),
        W3.astype(jnp.bfloat16))

    zeros1d = jnp.zeros((ACC,), dtype=jnp.float32)
    partials = _segment_scatter(t_all, coord_diff[:, 0], coord_diff[:, 1],
                                coord_diff[:, 2], row, zeros1d, e)

    agg = (partials[0] + partials[1]).reshape(-1)[:3 * n].reshape(n, 3)
    return coord + agg


# speedup vs baseline: 2.4878x; 1.1169x over previous
import functools

import jax
import jax.numpy as jnp
from jax import lax
from jax.experimental import pallas as pl
from jax.experimental.pallas import tpu as pltpu
from jax.experimental.pallas import tpu_sc as plsc

NC = 2
NS = 16
NW = NC * NS

COORDS_RANGE_OVER_NORM = 15.0 / 100.0


def _ab_body(h_ref, wa_ref, wb_ref, a_ref, b_ref):
    hb = h_ref[...].astype(jnp.bfloat16)
    dn = (((1,), (1,)), ((), ()))
    a_ref[...] = lax.dot_general(
        hb, wa_ref[...], dn, preferred_element_type=jnp.float32)
    b_ref[...] = lax.dot_general(
        hb, wb_ref[...], dn, preferred_element_type=jnp.float32)


def _node_ab(h, w1a, w1b):
    n, hdim = h.shape
    return pl.pallas_call(
        _ab_body,
        out_shape=(
            jax.ShapeDtypeStruct((n, hdim), jnp.float32),
            jax.ShapeDtypeStruct((n, hdim), jnp.float32),
        ),
    )(h, w1a, w1b)


IW = 80
TPC = 5
CG = IW * TPC
NSLOT = 2


def _gather_sum(a_t, b_t, row1d, col1d, e):
    epw = e // NW
    nchunks = epw // CG
    mesh = plsc.VectorSubcoreMesh(core_axis_name="c", subcore_axis_name="s")
    hdim = a_t.shape[1]

    @functools.partial(
        pl.kernel,
        out_type=jax.ShapeDtypeStruct((EPAD, hdim), jnp.float32),
        mesh=mesh,
        scratch_types=[
            pltpu.VMEM((NSLOT * CG,), jnp.int32),
            pltpu.VMEM((NSLOT * CG,), jnp.int32),
            pltpu.VMEM((NSLOT * CG, hdim), jnp.float32),
            pltpu.SemaphoreType.DMA,
            pltpu.SemaphoreType.DMA,
        ],
    )
    def k(a_hbm, b_hbm, row_hbm, col_hbm, g_hbm, rowv, colv, buf, semg, semw):
        wid = lax.axis_index("c") * NS + lax.axis_index("s")
        base0 = wid * epw

        wb = {}
        for c in range(nchunks):
            slot = c % NSLOT
            so = slot * CG
            if c >= NSLOT:
                wb.pop(c - NSLOT).wait()
            base = base0 + c * CG
            pltpu.sync_copy(row_hbm.at[pl.ds(base, CG)],
                            rowv.at[pl.ds(so, CG)])
            pltpu.sync_copy(col_hbm.at[pl.ds(base, CG)],
                            colv.at[pl.ds(so, CG)])
            gd = []
            for j in range(TPC):
                gd.append(pltpu.async_copy(
                    a_hbm.at[rowv.at[pl.ds(so + j * IW, IW)]],
                    buf.at[pl.ds(so + j * IW, IW)], semg))
            for d in gd:
                d.wait()
            gd = []
            for j in range(TPC):
                gd.append(pltpu.async_copy(
                    b_hbm.at[colv.at[pl.ds(so + j * IW, IW)]],
                    buf.at[pl.ds(so + j * IW, IW)], semg, add=True))
            for d in gd:
                d.wait()
            wb[c] = pltpu.async_copy(buf.at[pl.ds(so, CG)],
                                     g_hbm.at[pl.ds(base, CG)], semw)
        for c in sorted(wb):
            wb[c].wait()

    return k(a_t, b_t, row1d, col1d)


BE = 4096
EPAD = 327680


def _mlp_body(g_ref, ea_ref, w1c_ref, b1_ref, w2_ref, b2_ref, w3_ref, t_ref):
    dn = (((1,), (1,)), ((), ()))
    pre = g_ref[...] + lax.dot_general(
        ea_ref[...].astype(jnp.bfloat16), w1c_ref[...],
        (((0,), (1,)), ((), ())), preferred_element_type=jnp.float32)
    pre = (pre + b1_ref[...]).astype(jnp.bfloat16)
    x = pre * jax.nn.sigmoid(pre)
    pre2 = (lax.dot_general(
        x, w2_ref[...], dn, preferred_element_type=jnp.float32)
        + b2_ref[...]).astype(jnp.bfloat16)
    y = pre2 * jax.nn.sigmoid(pre2)
    s = lax.dot_general(
        w3_ref[...], y, dn, preferred_element_type=jnp.float32)
    t_ref[...] = (jnp.tanh(s) * COORDS_RANGE_OVER_NORM)[0]


def _edge_mlp(g, ea_t, w1c, b1r, w2, b2r, w3):
    epad, hdim = g.shape
    de = ea_t.shape[0]
    grid = (epad // BE,)

    return pl.pallas_call(
        _mlp_body,
        grid=grid,
        in_specs=[
            pl.BlockSpec((BE, hdim), lambda i: (i, 0)),
            pl.BlockSpec((de, BE), lambda i: (0, i)),
            pl.BlockSpec((hdim, de), lambda i: (0, 0)),
            pl.BlockSpec((1, hdim), lambda i: (0, 0)),
            pl.BlockSpec((hdim, hdim), lambda i: (0, 0)),
            pl.BlockSpec((1, hdim), lambda i: (0, 0)),
            pl.BlockSpec((1, hdim), lambda i: (0, 0)),
        ],
        out_specs=pl.BlockSpec((BE,), lambda i: (i,)),
        out_shape=jax.ShapeDtypeStruct((epad,), jnp.float32),
    )(g, ea_t, w1c, b1r, w2, b2r, w3)


ACC = 32768
C4 = 2000


def _segment_scatter(t_all, cd0, cd1, cd2, row1d, zeros1d, e):
    epw = e // NW
    mesh = plsc.VectorSubcoreMesh(core_axis_name="c", subcore_axis_name="s")
    sl = ACC // NS

    @functools.partial(
        pl.kernel,
        out_type=jax.ShapeDtypeStruct((NC, ACC // 128, 128), jnp.float32),
        mesh=mesh,
        scratch_types=[
            pltpu.VMEM((C4,), jnp.int32),
            pltpu.VMEM((C4,), jnp.float32),
            pltpu.VMEM((C4,), jnp.float32),
            pltpu.VMEM((C4,), jnp.float32),
            pltpu.VMEM((C4,), jnp.float32),
            pltpu.VMEM((ACC,), jnp.float32),
            pltpu.VMEM((NS, sl), jnp.float32),
            pltpu.VMEM((sl // 128, 128), jnp.float32),
            pltpu.VMEM_SHARED((NS, ACC), jnp.float32),
        ],
        compiler_params=pltpu.CompilerParams(needs_layout_passes=False),
    )
    def k(t_hbm, cd0_hbm, cd1_hbm, cd2_hbm, row_hbm, zero_hbm, out_hbm,
          rowv, tv, c0v, c1v, c2v, accl, buf2, res, stage):
        cid = lax.axis_index("c")
        sid = lax.axis_index("s")
        wid = cid * NS + sid

        pltpu.sync_copy(zero_hbm, accl)

        def chunk(i, carry):
            base = wid * epw + i * C4
            pltpu.sync_copy(row_hbm.at[pl.ds(base, C4)], rowv)
            pltpu.sync_copy(t_hbm.at[pl.ds(base, C4)], tv)
            pltpu.sync_copy(cd0_hbm.at[pl.ds(base, C4)], c0v)
            pltpu.sync_copy(cd1_hbm.at[pl.ds(base, C4)], c1v)
            pltpu.sync_copy(cd2_hbm.at[pl.ds(base, C4)], c2v)

            def grp(g, c2):
                rv = rowv[pl.ds(g * 16, 16)]
                th = tv[pl.ds(g * 16, 16)]
                f0 = rv * 3
                for d, cdv in enumerate((c0v, c1v, c2v)):
                    cd_d = cdv[pl.ds(g * 16, 16)]
                    plsc.addupdate_scatter(accl, [f0 + d], cd_d * th)
                return c2

            lax.fori_loop(0, C4 // 16, grp, 0)
            return carry

        lax.fori_loop(0, epw // C4, chunk, 0)

        pltpu.sync_copy(accl, stage.at[sid])
        plsc.subcore_barrier()
        pltpu.sync_copy(stage.at[:, pl.ds(sid * sl, sl)], buf2)

        def colsum_rows(jr, carry):
            for jc in range(8):
                j = jr * 8 + jc
                acc16 = buf2[0, pl.ds(j * 16, 16)]
                for r in range(1, NS):
                    acc16 = acc16 + buf2[r, pl.ds(j * 16, 16)]
                res[jr, pl.ds(jc * 16, 16)] = acc16
            return carry

        lax.fori_loop(0, sl // 128, colsum_rows, 0)
        pltpu.sync_copy(res, out_hbm.at[cid, pl.ds(sid * (sl // 128),
                                                   sl // 128)])

    return k(t_all, cd0, cd1, cd2, row1d, zeros1d)


def kernel(h, coord, edge_index, coord_diff, edge_attr, W1, b1, W2, b2, W3):
    n, hdim = h.shape
    e = edge_index.shape[1]

    w1a = W1[:, :hdim].astype(jnp.bfloat16)
    w1b = W1[:, hdim:2 * hdim].astype(jnp.bfloat16)
    w1c = W1[:, 2 * hdim:].astype(jnp.bfloat16)

    a_t, b_t = _node_ab(h, w1a, w1b)

    row = edge_index[0]
    col = edge_index[1]
    g = _gather_sum(a_t, b_t, row, col, e)

    ea_t = jnp.pad(edge_attr, ((0, EPAD - e), (0, 0))).T
    t_all = _edge_mlp(
        g, ea_t, w1c,
        b1.reshape(1, -1), W2.astype(jnp.bfloat16), b2.reshape(1, -1),
        W3.astype(jnp.bfloat16))

    zeros1d = jnp.zeros((ACC,), dtype=jnp.float32)
    partials = _segment_scatter(t_all, coord_diff[:, 0], coord_diff[:, 1],
                                coord_diff[:, 2], row, zeros1d, e)

    agg = (partials[0] + partials[1]).reshape(-1)[:3 * n].reshape(n, 3)
    return coord + agg


# speedup vs baseline: 2.6214x; 1.0537x over previous
import functools

import jax
import jax.numpy as jnp
from jax import lax
from jax.experimental import pallas as pl
from jax.experimental.pallas import tpu as pltpu
from jax.experimental.pallas import tpu_sc as plsc

NC = 2
NS = 16
NW = NC * NS

COORDS_RANGE_OVER_NORM = 15.0 / 100.0


def _ab_body(h_ref, wa_ref, wb_ref, a_ref, b_ref):
    hb = h_ref[...].astype(jnp.bfloat16)
    dn = (((1,), (1,)), ((), ()))
    a_ref[...] = lax.dot_general(
        hb, wa_ref[...], dn, preferred_element_type=jnp.float32)
    b_ref[...] = lax.dot_general(
        hb, wb_ref[...], dn, preferred_element_type=jnp.float32)


def _node_ab(h, w1a, w1b):
    n, hdim = h.shape
    return pl.pallas_call(
        _ab_body,
        out_shape=(
            jax.ShapeDtypeStruct((n, hdim), jnp.float32),
            jax.ShapeDtypeStruct((n, hdim), jnp.float32),
        ),
    )(h, w1a, w1b)


IW = 80
TPC = 5
CG = IW * TPC
NSLOT = 2


def _gather_sum(a_t, b_t, row1d, col1d, e):
    epw = e // NW
    nchunks = epw // CG
    mesh = plsc.VectorSubcoreMesh(core_axis_name="c", subcore_axis_name="s")
    hdim = a_t.shape[1]

    @functools.partial(
        pl.kernel,
        out_type=jax.ShapeDtypeStruct((EPAD, hdim), jnp.float32),
        mesh=mesh,
        scratch_types=[
            pltpu.VMEM((NSLOT * CG,), jnp.int32),
            pltpu.VMEM((NSLOT * CG,), jnp.int32),
            pltpu.VMEM((NSLOT * CG, hdim), jnp.float32),
            pltpu.SemaphoreType.DMA,
            pltpu.SemaphoreType.DMA,
        ],
    )
    def k(a_hbm, b_hbm, row_hbm, col_hbm, g_hbm, rowv, colv, buf, semg, semw):
        wid = lax.axis_index("c") * NS + lax.axis_index("s")
        base0 = wid * epw

        wb = {}
        for c in range(nchunks):
            slot = c % NSLOT
            so = slot * CG
            if c >= NSLOT:
                wb.pop(c - NSLOT).wait()
            base = base0 + c * CG
            pltpu.sync_copy(row_hbm.at[pl.ds(base, CG)],
                            rowv.at[pl.ds(so, CG)])
            pltpu.sync_copy(col_hbm.at[pl.ds(base, CG)],
                            colv.at[pl.ds(so, CG)])
            gd = []
            for j in range(TPC):
                gd.append(pltpu.async_copy(
                    a_hbm.at[rowv.at[pl.ds(so + j * IW, IW)]],
                    buf.at[pl.ds(so + j * IW, IW)], semg))
            for d in gd:
                d.wait()
            gd = []
            for j in range(TPC):
                gd.append(pltpu.async_copy(
                    b_hbm.at[colv.at[pl.ds(so + j * IW, IW)]],
                    buf.at[pl.ds(so + j * IW, IW)], semg, add=True))
            for d in gd:
                d.wait()
            wb[c] = pltpu.async_copy(buf.at[pl.ds(so, CG)],
                                     g_hbm.at[pl.ds(base, CG)], semw)
        for c in sorted(wb):
            wb[c].wait()

    return k(a_t, b_t, row1d, col1d)


BE = 8192
EPAD = 327680


def _mlp_body(g_ref, ea_ref, w1c_ref, b1_ref, w2_ref, b2_ref, w3_ref, t_ref):
    dn = (((1,), (1,)), ((), ()))
    pre = g_ref[...] + lax.dot_general(
        ea_ref[...].astype(jnp.bfloat16), w1c_ref[...],
        (((0,), (1,)), ((), ())), preferred_element_type=jnp.float32)
    pre = (pre + b1_ref[...]).astype(jnp.bfloat16)
    x = pre * jax.nn.sigmoid(pre)
    pre2 = (lax.dot_general(
        x, w2_ref[...], dn, preferred_element_type=jnp.float32)
        + b2_ref[...]).astype(jnp.bfloat16)
    y = pre2 * jax.nn.sigmoid(pre2)
    s = lax.dot_general(
        w3_ref[...], y, dn, preferred_element_type=jnp.float32)
    t_ref[...] = (jnp.tanh(s) * COORDS_RANGE_OVER_NORM)[0]


def _edge_mlp(g, ea_t, w1c, b1r, w2, b2r, w3):
    epad, hdim = g.shape
    de = ea_t.shape[0]
    grid = (epad // BE,)

    return pl.pallas_call(
        _mlp_body,
        grid=grid,
        in_specs=[
            pl.BlockSpec((BE, hdim), lambda i: (i, 0)),
            pl.BlockSpec((de, BE), lambda i: (0, i)),
            pl.BlockSpec((hdim, de), lambda i: (0, 0)),
            pl.BlockSpec((1, hdim), lambda i: (0, 0)),
            pl.BlockSpec((hdim, hdim), lambda i: (0, 0)),
            pl.BlockSpec((1, hdim), lambda i: (0, 0)),
            pl.BlockSpec((1, hdim), lambda i: (0, 0)),
        ],
        out_specs=pl.BlockSpec((BE,), lambda i: (i,)),
        out_shape=jax.ShapeDtypeStruct((epad,), jnp.float32),
    )(g, ea_t, w1c, b1r, w2, b2r, w3)


ACC = 32768
C4 = 2000


def _segment_scatter(t_all, cd0, cd1, cd2, row1d, zeros1d, e):
    epw = e // NW
    mesh = plsc.VectorSubcoreMesh(core_axis_name="c", subcore_axis_name="s")
    sl = ACC // NS

    @functools.partial(
        pl.kernel,
        out_type=jax.ShapeDtypeStruct((NC, ACC // 128, 128), jnp.float32),
        mesh=mesh,
        scratch_types=[
            pltpu.VMEM((C4,), jnp.int32),
            pltpu.VMEM((C4,), jnp.float32),
            pltpu.VMEM((C4,), jnp.float32),
            pltpu.VMEM((C4,), jnp.float32),
            pltpu.VMEM((C4,), jnp.float32),
            pltpu.VMEM((ACC,), jnp.float32),
            pltpu.VMEM((NS, sl), jnp.float32),
            pltpu.VMEM((sl // 128, 128), jnp.float32),
            pltpu.VMEM_SHARED((NS, ACC), jnp.float32),
        ],
        compiler_params=pltpu.CompilerParams(needs_layout_passes=False),
    )
    def k(t_hbm, cd0_hbm, cd1_hbm, cd2_hbm, row_hbm, zero_hbm, out_hbm,
          rowv, tv, c0v, c1v, c2v, accl, buf2, res, stage):
        cid = lax.axis_index("c")
        sid = lax.axis_index("s")
        wid = cid * NS + sid

        pltpu.sync_copy(zero_hbm, accl)

        def chunk(i, carry):
            base = wid * epw + i * C4
            pltpu.sync_copy(row_hbm.at[pl.ds(base, C4)], rowv)
            pltpu.sync_copy(t_hbm.at[pl.ds(base, C4)], tv)
            pltpu.sync_copy(cd0_hbm.at[pl.ds(base, C4)], c0v)
            pltpu.sync_copy(cd1_hbm.at[pl.ds(base, C4)], c1v)
            pltpu.sync_copy(cd2_hbm.at[pl.ds(base, C4)], c2v)

            def grp(g, c2):
                rv = rowv[pl.ds(g * 16, 16)]
                th = tv[pl.ds(g * 16, 16)]
                f0 = rv * 3
                for d, cdv in enumerate((c0v, c1v, c2v)):
                    cd_d = cdv[pl.ds(g * 16, 16)]
                    plsc.addupdate_scatter(accl, [f0 + d], cd_d * th)
                return c2

            lax.fori_loop(0, C4 // 16, grp, 0)
            return carry

        lax.fori_loop(0, epw // C4, chunk, 0)

        pltpu.sync_copy(accl, stage.at[sid])
        plsc.subcore_barrier()
        pltpu.sync_copy(stage.at[:, pl.ds(sid * sl, sl)], buf2)

        def colsum_rows(jr, carry):
            for jc in range(8):
                j = jr * 8 + jc
                acc16 = buf2[0, pl.ds(j * 16, 16)]
                for r in range(1, NS):
                    acc16 = acc16 + buf2[r, pl.ds(j * 16, 16)]
                res[jr, pl.ds(jc * 16, 16)] = acc16
            return carry

        lax.fori_loop(0, sl // 128, colsum_rows, 0)
        pltpu.sync_copy(res, out_hbm.at[cid, pl.ds(sid * (sl // 128),
                                                   sl // 128)])

    return k(t_all, cd0, cd1, cd2, row1d, zeros1d)


def kernel(h, coord, edge_index, coord_diff, edge_attr, W1, b1, W2, b2, W3):
    n, hdim = h.shape
    e = edge_index.shape[1]

    w1a = W1[:, :hdim].astype(jnp.bfloat16)
    w1b = W1[:, hdim:2 * hdim].astype(jnp.bfloat16)
    w1c = W1[:, 2 * hdim:].astype(jnp.bfloat16)

    a_t, b_t = _node_ab(h, w1a, w1b)

    row = edge_index[0]
    col = edge_index[1]
    g = _gather_sum(a_t, b_t, row, col, e)

    ea_t = jnp.pad(edge_attr, ((0, EPAD - e), (0, 0))).T
    t_all = _edge_mlp(
        g, ea_t, w1c,
        b1.reshape(1, -1), W2.astype(jnp.bfloat16), b2.reshape(1, -1),
        W3.astype(jnp.bfloat16))

    zeros1d = jnp.zeros((ACC,), dtype=jnp.float32)
    partials = _segment_scatter(t_all, coord_diff[:, 0], coord_diff[:, 1],
                                coord_diff[:, 2], row, zeros1d, e)

    agg = (partials[0] + partials[1]).reshape(-1)[:3 * n].reshape(n, 3)
    return coord + agg


# speedup vs baseline: 2.8361x; 1.0819x over previous
import functools

import jax
import jax.numpy as jnp
from jax import lax
from jax.experimental import pallas as pl
from jax.experimental.pallas import tpu as pltpu
from jax.experimental.pallas import tpu_sc as plsc

NC = 2
NS = 16
NW = NC * NS

COORDS_RANGE_OVER_NORM = 15.0 / 100.0


def _ab_body(h_ref, wa_ref, wb_ref, a_ref, b_ref):
    hb = h_ref[...].astype(jnp.bfloat16)
    dn = (((1,), (1,)), ((), ()))
    a_ref[...] = lax.dot_general(
        hb, wa_ref[...], dn, preferred_element_type=jnp.float32)
    b_ref[...] = lax.dot_general(
        hb, wb_ref[...], dn, preferred_element_type=jnp.float32)


def _node_ab(h, w1a, w1b):
    n, hdim = h.shape
    return pl.pallas_call(
        _ab_body,
        out_shape=(
            jax.ShapeDtypeStruct((n, hdim), jnp.float32),
            jax.ShapeDtypeStruct((n, hdim), jnp.float32),
        ),
    )(h, w1a, w1b)


IW = 80
TPC = 5
CG = IW * TPC
NSLOT = 2


def _gather_sum(a_t, b_t, row1d, col1d, e0, epw, gpad):
    sizes = [CG] * (epw // CG)
    if epw % CG:
        sizes.append(epw % CG)
    mesh = plsc.VectorSubcoreMesh(core_axis_name="c", subcore_axis_name="s")
    hdim = a_t.shape[1]

    @functools.partial(
        pl.kernel,
        out_type=jax.ShapeDtypeStruct((gpad, hdim), jnp.float32),
        mesh=mesh,
        scratch_types=[
            pltpu.VMEM((NSLOT * CG,), jnp.int32),
            pltpu.VMEM((NSLOT * CG,), jnp.int32),
            pltpu.VMEM((NSLOT * CG, hdim), jnp.float32),
            pltpu.SemaphoreType.DMA,
            pltpu.SemaphoreType.DMA,
        ],
    )
    def k(a_hbm, b_hbm, row_hbm, col_hbm, g_hbm, rowv, colv, buf, semg, semw):
        wid = lax.axis_index("c") * NS + lax.axis_index("s")
        lbase0 = wid * epw

        wb = {}
        loff = 0
        for c, sz in enumerate(sizes):
            slot = c % NSLOT
            so = slot * CG
            if c >= NSLOT:
                wb.pop(c - NSLOT).wait()
            lbase = lbase0 + loff
            pltpu.sync_copy(row_hbm.at[pl.ds(e0 + lbase, sz)],
                            rowv.at[pl.ds(so, sz)])
            pltpu.sync_copy(col_hbm.at[pl.ds(e0 + lbase, sz)],
                            colv.at[pl.ds(so, sz)])
            for idxv in (rowv, colv):
                gd = []
                for j in range(sz // IW):
                    gd.append(pltpu.async_copy(
                        a_hbm.at[idxv.at[pl.ds(so + j * IW, IW)]] if idxv is rowv
                        else b_hbm.at[idxv.at[pl.ds(so + j * IW, IW)]],
                        buf.at[pl.ds(so + j * IW, IW)], semg,
                        add=(idxv is colv)))
                for d in gd:
                    d.wait()
            wb[c] = pltpu.async_copy(buf.at[pl.ds(so, sz)],
                                     g_hbm.at[pl.ds(lbase, sz)], semw)
            loff += sz
        for c in sorted(wb):
            wb[c].wait()

    return k(a_t, b_t, row1d, col1d)


BE = 8192
EPAD = 327680


def _mlp_body(g_ref, ea_ref, w1c_ref, b1_ref, w2_ref, b2_ref, w3_ref, t_ref):
    dn = (((1,), (1,)), ((), ()))
    pre = g_ref[...] + lax.dot_general(
        ea_ref[...].astype(jnp.bfloat16), w1c_ref[...],
        (((0,), (1,)), ((), ())), preferred_element_type=jnp.float32)
    pre = (pre + b1_ref[...]).astype(jnp.bfloat16)
    x = pre * jax.nn.sigmoid(pre)
    pre2 = (lax.dot_general(
        x, w2_ref[...], dn, preferred_element_type=jnp.float32)
        + b2_ref[...]).astype(jnp.bfloat16)
    y = pre2 * jax.nn.sigmoid(pre2)
    s = lax.dot_general(
        w3_ref[...], y, dn, preferred_element_type=jnp.float32)
    t_ref[...] = (jnp.tanh(s) * COORDS_RANGE_OVER_NORM)[0]


def _edge_mlp(g, ea_t, blk0, w1c, b1r, w2, b2r, w3):
    gpad, hdim = g.shape
    de = ea_t.shape[0]
    grid = (gpad // BE,)

    return pl.pallas_call(
        _mlp_body,
        grid=grid,
        in_specs=[
            pl.BlockSpec((BE, hdim), lambda i: (i, 0)),
            pl.BlockSpec((de, BE), lambda i: (0, blk0 + i)),
            pl.BlockSpec((hdim, de), lambda i: (0, 0)),
            pl.BlockSpec((1, hdim), lambda i: (0, 0)),
            pl.BlockSpec((hdim, hdim), lambda i: (0, 0)),
            pl.BlockSpec((1, hdim), lambda i: (0, 0)),
            pl.BlockSpec((1, hdim), lambda i: (0, 0)),
        ],
        out_specs=pl.BlockSpec((BE,), lambda i: (i,)),
        out_shape=jax.ShapeDtypeStruct((gpad,), jnp.float32),
    )(g, ea_t, w1c, b1r, w2, b2r, w3)


ACC = 32768
C4 = 2000


def _segment_scatter(t_loc, cd0, cd1, cd2, row1d, zeros1d, e0, epw):
    sizes = [C4] * (epw // C4)
    if epw % C4:
        sizes.append(epw % C4)
    mesh = plsc.VectorSubcoreMesh(core_axis_name="c", subcore_axis_name="s")
    sl = ACC // NS

    @functools.partial(
        pl.kernel,
        out_type=jax.ShapeDtypeStruct((NC, ACC // 128, 128), jnp.float32),
        mesh=mesh,
        scratch_types=[
            pltpu.VMEM((C4,), jnp.int32),
            pltpu.VMEM((C4,), jnp.float32),
            pltpu.VMEM((C4,), jnp.float32),
            pltpu.VMEM((C4,), jnp.float32),
            pltpu.VMEM((C4,), jnp.float32),
            pltpu.VMEM((ACC,), jnp.float32),
            pltpu.VMEM((NS, sl), jnp.float32),
            pltpu.VMEM((sl // 128, 128), jnp.float32),
            pltpu.VMEM_SHARED((NS, ACC), jnp.float32),
        ],
        compiler_params=pltpu.CompilerParams(needs_layout_passes=False),
    )
    def k(t_hbm, cd0_hbm, cd1_hbm, cd2_hbm, row_hbm, zero_hbm, out_hbm,
          rowv, tv, c0v, c1v, c2v, accl, buf2, res, stage):
        cid = lax.axis_index("c")
        sid = lax.axis_index("s")
        wid = cid * NS + sid

        pltpu.sync_copy(zero_hbm, accl)

        loff = 0
        for sz in sizes:
            lbase = wid * epw + loff
            pltpu.sync_copy(row_hbm.at[pl.ds(e0 + lbase, sz)],
                            rowv.at[pl.ds(0, sz)])
            pltpu.sync_copy(t_hbm.at[pl.ds(lbase, sz)], tv.at[pl.ds(0, sz)])
            pltpu.sync_copy(cd0_hbm.at[pl.ds(e0 + lbase, sz)],
                            c0v.at[pl.ds(0, sz)])
            pltpu.sync_copy(cd1_hbm.at[pl.ds(e0 + lbase, sz)],
                            c1v.at[pl.ds(0, sz)])
            pltpu.sync_copy(cd2_hbm.at[pl.ds(e0 + lbase, sz)],
                            c2v.at[pl.ds(0, sz)])

            def grp(g, c2):
                rv = rowv[pl.ds(g * 16, 16)]
                th = tv[pl.ds(g * 16, 16)]
                f0 = rv * 3
                for d, cdv in enumerate((c0v, c1v, c2v)):
                    cd_d = cdv[pl.ds(g * 16, 16)]
                    plsc.addupdate_scatter(accl, [f0 + d], cd_d * th)
                return c2

            lax.fori_loop(0, sz // 16, grp, 0)
            loff += sz

        pltpu.sync_copy(accl, stage.at[sid])
        plsc.subcore_barrier()
        pltpu.sync_copy(stage.at[:, pl.ds(sid * sl, sl)], buf2)

        def colsum_rows(jr, carry):
            for jc in range(8):
                j = jr * 8 + jc
                acc16 = buf2[0, pl.ds(j * 16, 16)]
                for r in range(1, NS):
                    acc16 = acc16 + buf2[r, pl.ds(j * 16, 16)]
                res[jr, pl.ds(jc * 16, 16)] = acc16
            return carry

        lax.fori_loop(0, sl // 128, colsum_rows, 0)
        pltpu.sync_copy(res, out_hbm.at[cid, pl.ds(sid * (sl // 128),
                                                   sl // 128)])

    return k(t_loc, cd0, cd1, cd2, row1d, zeros1d)


def kernel(h, coord, edge_index, coord_diff, edge_attr, W1, b1, W2, b2, W3):
    n, hdim = h.shape
    e = edge_index.shape[1]

    w1a = W1[:, :hdim].astype(jnp.bfloat16)
    w1b = W1[:, hdim:2 * hdim].astype(jnp.bfloat16)
    w1c = W1[:, 2 * hdim:].astype(jnp.bfloat16)

    a_t, b_t = _node_ab(h, w1a, w1b)

    row = edge_index[0]
    col = edge_index[1]

    h0 = 163840
    gpad = 163840
    epw_a = h0 // NW
    epw_b = (e - h0) // NW

    ea_t = jnp.pad(edge_attr, ((0, EPAD - e), (0, 0))).T
    b1r = b1.reshape(1, -1)
    b2r = b2.reshape(1, -1)
    w2c = W2.astype(jnp.bfloat16)
    w3c = W3.astype(jnp.bfloat16)
    zeros1d = jnp.zeros((ACC,), dtype=jnp.float32)
    cd0, cd1, cd2 = coord_diff[:, 0], coord_diff[:, 1], coord_diff[:, 2]

    g_a = _gather_sum(a_t, b_t, row, col, 0, epw_a, gpad)
    g_b = _gather_sum(a_t, b_t, row, col, h0, epw_b, gpad)

    t_a = _edge_mlp(g_a, ea_t, 0, w1c, b1r, w2c, b2r, w3c)
    t_b = _edge_mlp(g_b, ea_t, h0 // BE, w1c, b1r, w2c, b2r, w3c)

    pa = _segment_scatter(t_a, cd0, cd1, cd2, row, zeros1d, 0, epw_a)
    pb = _segment_scatter(t_b, cd0, cd1, cd2, row, zeros1d, h0, epw_b)

    agg = ((pa[0] + pa[1]) + (pb[0] + pb[1])).reshape(-1)[:3 * n].reshape(n, 3)
    return coord + agg


# speedup vs baseline: 2.9442x; 1.0381x over previous
import functools

import jax
import jax.numpy as jnp
from jax import lax
from jax.experimental import pallas as pl
from jax.experimental.pallas import tpu as pltpu
from jax.experimental.pallas import tpu_sc as plsc

NC = 2
NS = 16
NW = NC * NS

COORDS_RANGE_OVER_NORM = 15.0 / 100.0


def _ab_body(h_ref, wa_ref, wb_ref, a_ref, b_ref):
    hb = h_ref[...].astype(jnp.bfloat16)
    dn = (((1,), (1,)), ((), ()))
    a_ref[...] = lax.dot_general(
        hb, wa_ref[...], dn, preferred_element_type=jnp.float32)
    b_ref[...] = lax.dot_general(
        hb, wb_ref[...], dn, preferred_element_type=jnp.float32)


def _node_ab(h, w1a, w1b):
    n, hdim = h.shape
    return pl.pallas_call(
        _ab_body,
        out_shape=(
            jax.ShapeDtypeStruct((n, hdim), jnp.float32),
            jax.ShapeDtypeStruct((n, hdim), jnp.float32),
        ),
    )(h, w1a, w1b)


IW = 80
TPC = 5
CG = IW * TPC
NSLOT = 2


def _gather_sum(a_t, b_t, row1d, col1d, e0, epw, gpad):
    sizes = [CG] * (epw // CG)
    if epw % CG:
        sizes.append(epw % CG)
    mesh = plsc.VectorSubcoreMesh(core_axis_name="c", subcore_axis_name="s")
    hdim = a_t.shape[1]

    @functools.partial(
        pl.kernel,
        out_type=jax.ShapeDtypeStruct((gpad, hdim), jnp.float32),
        mesh=mesh,
        scratch_types=[
            pltpu.VMEM((NSLOT * CG,), jnp.int32),
            pltpu.VMEM((NSLOT * CG,), jnp.int32),
            pltpu.VMEM((NSLOT * CG, hdim), jnp.float32),
            pltpu.SemaphoreType.DMA,
            pltpu.SemaphoreType.DMA,
            pltpu.SemaphoreType.DMA,
        ],
    )
    def k(a_hbm, b_hbm, row_hbm, col_hbm, g_hbm, rowv, colv, buf,
          sema, semb, semw):
        wid = lax.axis_index("c") * NS + lax.axis_index("s")
        lbase0 = wid * epw
        n = len(sizes)
        offs = [0]
        for sz in sizes:
            offs.append(offs[-1] + sz)

        def load_idx(c):
            so = (c % NSLOT) * CG
            sz = sizes[c]
            lbase = lbase0 + offs[c]
            pltpu.sync_copy(row_hbm.at[pl.ds(e0 + lbase, sz)],
                            rowv.at[pl.ds(so, sz)])
            pltpu.sync_copy(col_hbm.at[pl.ds(e0 + lbase, sz)],
                            colv.at[pl.ds(so, sz)])

        def gather(c, idxv, tbl, sem, add):
            so = (c % NSLOT) * CG
            return [
                pltpu.async_copy(
                    tbl.at[idxv.at[pl.ds(so + j * IW, IW)]],
                    buf.at[pl.ds(so + j * IW, IW)], sem, add=add)
                for j in range(sizes[c] // IW)
            ]

        wb = {}
        load_idx(0)
        ga = gather(0, rowv, a_hbm, sema, False)
        for c in range(n):
            for d in ga:
                d.wait()
            gb = gather(c, colv, b_hbm, semb, True)
            if c + 1 < n:
                if c - 1 >= 0:
                    wb.pop(c - 1).wait()
                load_idx(c + 1)
                ga = gather(c + 1, rowv, a_hbm, sema, False)
            for d in gb:
                d.wait()
            so = (c % NSLOT) * CG
            wb[c] = pltpu.async_copy(
                buf.at[pl.ds(so, sizes[c])],
                g_hbm.at[pl.ds(lbase0 + offs[c], sizes[c])], semw)
        for c in sorted(wb):
            wb[c].wait()

    return k(a_t, b_t, row1d, col1d)


BE = 8192
EPAD = 327680


def _mlp_body(g_ref, ea_ref, w1c_ref, b1_ref, w2_ref, b2_ref, w3_ref, t_ref):
    dn = (((1,), (1,)), ((), ()))
    pre = g_ref[...] + lax.dot_general(
        ea_ref[...].astype(jnp.bfloat16), w1c_ref[...],
        (((0,), (1,)), ((), ())), preferred_element_type=jnp.float32)
    pre = (pre + b1_ref[...]).astype(jnp.bfloat16)
    x = pre * jax.nn.sigmoid(pre)
    pre2 = (lax.dot_general(
        x, w2_ref[...], dn, preferred_element_type=jnp.float32)
        + b2_ref[...]).astype(jnp.bfloat16)
    y = pre2 * jax.nn.sigmoid(pre2)
    s = lax.dot_general(
        w3_ref[...], y, dn, preferred_element_type=jnp.float32)
    t_ref[...] = (jnp.tanh(s) * COORDS_RANGE_OVER_NORM)[0]


def _edge_mlp(g, ea_t, blk0, w1c, b1r, w2, b2r, w3):
    gpad, hdim = g.shape
    de = ea_t.shape[0]
    grid = (gpad // BE,)

    return pl.pallas_call(
        _mlp_body,
        grid=grid,
        in_specs=[
            pl.BlockSpec((BE, hdim), lambda i: (i, 0)),
            pl.BlockSpec((de, BE), lambda i: (0, blk0 + i)),
            pl.BlockSpec((hdim, de), lambda i: (0, 0)),
            pl.BlockSpec((1, hdim), lambda i: (0, 0)),
            pl.BlockSpec((hdim, hdim), lambda i: (0, 0)),
            pl.BlockSpec((1, hdim), lambda i: (0, 0)),
            pl.BlockSpec((1, hdim), lambda i: (0, 0)),
        ],
        out_specs=pl.BlockSpec((BE,), lambda i: (i,)),
        out_shape=jax.ShapeDtypeStruct((gpad,), jnp.float32),
    )(g, ea_t, w1c, b1r, w2, b2r, w3)


ACC = 32768
C4 = 2000


def _segment_scatter(t_loc, cd0, cd1, cd2, row1d, zeros1d, e0, epw):
    sizes = [C4] * (epw // C4)
    if epw % C4:
        sizes.append(epw % C4)
    mesh = plsc.VectorSubcoreMesh(core_axis_name="c", subcore_axis_name="s")
    sl = ACC // NS

    @functools.partial(
        pl.kernel,
        out_type=jax.ShapeDtypeStruct((NC, ACC // 128, 128), jnp.float32),
        mesh=mesh,
        scratch_types=[
            pltpu.VMEM((C4,), jnp.int32),
            pltpu.VMEM((C4,), jnp.float32),
            pltpu.VMEM((C4,), jnp.float32),
            pltpu.VMEM((C4,), jnp.float32),
            pltpu.VMEM((C4,), jnp.float32),
            pltpu.VMEM((ACC,), jnp.float32),
            pltpu.VMEM((NS, sl), jnp.float32),
            pltpu.VMEM((sl // 128, 128), jnp.float32),
            pltpu.VMEM_SHARED((NS, ACC), jnp.float32),
        ],
        compiler_params=pltpu.CompilerParams(needs_layout_passes=False),
    )
    def k(t_hbm, cd0_hbm, cd1_hbm, cd2_hbm, row_hbm, zero_hbm, out_hbm,
          rowv, tv, c0v, c1v, c2v, accl, buf2, res, stage):
        cid = lax.axis_index("c")
        sid = lax.axis_index("s")
        wid = cid * NS + sid

        pltpu.sync_copy(zero_hbm, accl)

        loff = 0
        for sz in sizes:
            lbase = wid * epw + loff
            pltpu.sync_copy(row_hbm.at[pl.ds(e0 + lbase, sz)],
                            rowv.at[pl.ds(0, sz)])
            pltpu.sync_copy(t_hbm.at[pl.ds(lbase, sz)], tv.at[pl.ds(0, sz)])
            pltpu.sync_copy(cd0_hbm.at[pl.ds(e0 + lbase, sz)],
                            c0v.at[pl.ds(0, sz)])
            pltpu.sync_copy(cd1_hbm.at[pl.ds(e0 + lbase, sz)],
                            c1v.at[pl.ds(0, sz)])
            pltpu.sync_copy(cd2_hbm.at[pl.ds(e0 + lbase, sz)],
                            c2v.at[pl.ds(0, sz)])

            def grp(g, c2):
                rv = rowv[pl.ds(g * 16, 16)]
                th = tv[pl.ds(g * 16, 16)]
                f0 = rv * 3
                for d, cdv in enumerate((c0v, c1v, c2v)):
                    cd_d = cdv[pl.ds(g * 16, 16)]
                    plsc.addupdate_scatter(accl, [f0 + d], cd_d * th)
                return c2

            lax.fori_loop(0, sz // 16, grp, 0)
            loff += sz

        pltpu.sync_copy(accl, stage.at[sid])
        plsc.subcore_barrier()
        pltpu.sync_copy(stage.at[:, pl.ds(sid * sl, sl)], buf2)

        def colsum_rows(jr, carry):
            for jc in range(8):
                j = jr * 8 + jc
                acc16 = buf2[0, pl.ds(j * 16, 16)]
                for r in range(1, NS):
                    acc16 = acc16 + buf2[r, pl.ds(j * 16, 16)]
                res[jr, pl.ds(jc * 16, 16)] = acc16
            return carry

        lax.fori_loop(0, sl // 128, colsum_rows, 0)
        pltpu.sync_copy(res, out_hbm.at[cid, pl.ds(sid * (sl // 128),
                                                   sl // 128)])

    return k(t_loc, cd0, cd1, cd2, row1d, zeros1d)


def kernel(h, coord, edge_index, coord_diff, edge_attr, W1, b1, W2, b2, W3):
    n, hdim = h.shape
    e = edge_index.shape[1]

    w1a = W1[:, :hdim].astype(jnp.bfloat16)
    w1b = W1[:, hdim:2 * hdim].astype(jnp.bfloat16)
    w1c = W1[:, 2 * hdim:].astype(jnp.bfloat16)

    a_t, b_t = _node_ab(h, w1a, w1b)

    row = edge_index[0]
    col = edge_index[1]

    h0 = 163840
    gpad = 163840
    epw_a = h0 // NW
    epw_b = (e - h0) // NW

    ea_t = jnp.pad(edge_attr, ((0, EPAD - e), (0, 0))).T
    b1r = b1.reshape(1, -1)
    b2r = b2.reshape(1, -1)
    w2c = W2.astype(jnp.bfloat16)
    w3c = W3.astype(jnp.bfloat16)
    zeros1d = jnp.zeros((ACC,), dtype=jnp.float32)
    cd0, cd1, cd2 = coord_diff[:, 0], coord_diff[:, 1], coord_diff[:, 2]

    g_a = _gather_sum(a_t, b_t, row, col, 0, epw_a, gpad)
    g_b = _gather_sum(a_t, b_t, row, col, h0, epw_b, gpad)

    t_a = _edge_mlp(g_a, ea_t, 0, w1c, b1r, w2c, b2r, w3c)
    t_b = _edge_mlp(g_b, ea_t, h0 // BE, w1c, b1r, w2c, b2r, w3c)

    pa = _segment_scatter(t_a, cd0, cd1, cd2, row, zeros1d, 0, epw_a)
    pb = _segment_scatter(t_b, cd0, cd1, cd2, row, zeros1d, h0, epw_b)

    agg = ((pa[0] + pa[1]) + (pb[0] + pb[1])).reshape(-1)[:3 * n].reshape(n, 3)
    return coord + agg


# speedup vs baseline: 3.0361x; 1.0312x over previous
import functools

import jax
import jax.numpy as jnp
from jax import lax
from jax.experimental import pallas as pl
from jax.experimental.pallas import tpu as pltpu
from jax.experimental.pallas import tpu_sc as plsc

NC = 2
NS = 16
NW = NC * NS

COORDS_RANGE_OVER_NORM = 15.0 / 100.0


def _ab_body(h_ref, wa_ref, wb_ref, a_ref, b_ref):
    hb = h_ref[...].astype(jnp.bfloat16)
    dn = (((1,), (1,)), ((), ()))
    a_ref[...] = lax.dot_general(
        hb, wa_ref[...], dn, preferred_element_type=jnp.float32)
    b_ref[...] = lax.dot_general(
        hb, wb_ref[...], dn, preferred_element_type=jnp.float32)


def _node_ab(h, w1a, w1b):
    n, hdim = h.shape
    return pl.pallas_call(
        _ab_body,
        out_shape=(
            jax.ShapeDtypeStruct((n, hdim), jnp.float32),
            jax.ShapeDtypeStruct((n, hdim), jnp.float32),
        ),
    )(h, w1a, w1b)


IW = 80
TPC = 5
CG = IW * TPC
NSLOT = 2


def _gather_sum(a_t, b_t, row1d, col1d, e0, epw, gpad):
    sizes = [CG] * (epw // CG)
    if epw % CG:
        sizes.append(epw % CG)
    mesh = plsc.VectorSubcoreMesh(core_axis_name="c", subcore_axis_name="s")
    hdim = a_t.shape[1]

    @functools.partial(
        pl.kernel,
        out_type=jax.ShapeDtypeStruct((gpad, hdim), jnp.float32),
        mesh=mesh,
        scratch_types=[
            pltpu.VMEM((NSLOT * CG,), jnp.int32),
            pltpu.VMEM((NSLOT * CG,), jnp.int32),
            pltpu.VMEM((NSLOT * CG, hdim), jnp.float32),
            pltpu.SemaphoreType.DMA,
            pltpu.SemaphoreType.DMA,
            pltpu.SemaphoreType.DMA,
        ],
    )
    def k(a_hbm, b_hbm, row_hbm, col_hbm, g_hbm, rowv, colv, buf,
          sema, semb, semw):
        wid = lax.axis_index("c") * NS + lax.axis_index("s")
        lbase0 = wid * epw
        n = len(sizes)
        offs = [0]
        for sz in sizes:
            offs.append(offs[-1] + sz)

        def load_idx(c):
            so = (c % NSLOT) * CG
            sz = sizes[c]
            lbase = lbase0 + offs[c]
            pltpu.sync_copy(row_hbm.at[pl.ds(e0 + lbase, sz)],
                            rowv.at[pl.ds(so, sz)])
            pltpu.sync_copy(col_hbm.at[pl.ds(e0 + lbase, sz)],
                            colv.at[pl.ds(so, sz)])

        def gather(c, idxv, tbl, sem, add):
            so = (c % NSLOT) * CG
            return [
                pltpu.async_copy(
                    tbl.at[idxv.at[pl.ds(so + j * IW, IW)]],
                    buf.at[pl.ds(so + j * IW, IW)], sem, add=add)
                for j in range(sizes[c] // IW)
            ]

        wb = {}
        load_idx(0)
        ga = gather(0, rowv, a_hbm, sema, False)
        for c in range(n):
            for d in ga:
                d.wait()
            gb = gather(c, colv, b_hbm, semb, True)
            if c + 1 < n:
                if c - 1 >= 0:
                    wb.pop(c - 1).wait()
                load_idx(c + 1)
                ga = gather(c + 1, rowv, a_hbm, sema, False)
            for d in gb:
                d.wait()
            so = (c % NSLOT) * CG
            wb[c] = pltpu.async_copy(
                buf.at[pl.ds(so, sizes[c])],
                g_hbm.at[pl.ds(lbase0 + offs[c], sizes[c])], semw)
        for c in sorted(wb):
            wb[c].wait()

    return k(a_t, b_t, row1d, col1d)


BE = 8192
EPAD = 327680


def _mlp_body(g_ref, ea_ref, w1c_ref, b1_ref, w2_ref, b2_ref, w3_ref, t_ref):
    dn = (((1,), (1,)), ((), ()))
    pre = g_ref[...] + lax.dot_general(
        ea_ref[...].astype(jnp.bfloat16), w1c_ref[...],
        (((0,), (1,)), ((), ())), preferred_element_type=jnp.float32)
    pre = (pre + b1_ref[...]).astype(jnp.bfloat16)
    x = pre * jax.nn.sigmoid(pre)
    pre2 = (lax.dot_general(
        x, w2_ref[...], dn, preferred_element_type=jnp.float32)
        + b2_ref[...]).astype(jnp.bfloat16)
    y = pre2 * jax.nn.sigmoid(pre2)
    s = lax.dot_general(
        w3_ref[...], y, dn, preferred_element_type=jnp.float32)
    t_ref[...] = (jnp.tanh(s) * COORDS_RANGE_OVER_NORM)[0]


def _edge_mlp(g, ea_t, blk0, w1c, b1r, w2, b2r, w3):
    gpad, hdim = g.shape
    de = ea_t.shape[0]
    grid = (gpad // BE,)

    return pl.pallas_call(
        _mlp_body,
        grid=grid,
        in_specs=[
            pl.BlockSpec((BE, hdim), lambda i: (i, 0)),
            pl.BlockSpec((de, BE), lambda i: (0, blk0 + i)),
            pl.BlockSpec((hdim, de), lambda i: (0, 0)),
            pl.BlockSpec((1, hdim), lambda i: (0, 0)),
            pl.BlockSpec((hdim, hdim), lambda i: (0, 0)),
            pl.BlockSpec((1, hdim), lambda i: (0, 0)),
            pl.BlockSpec((1, hdim), lambda i: (0, 0)),
        ],
        out_specs=pl.BlockSpec((BE,), lambda i: (i,)),
        out_shape=jax.ShapeDtypeStruct((gpad,), jnp.float32),
    )(g, ea_t, w1c, b1r, w2, b2r, w3)


ACC = 32768
C4 = 2000


def _segment_scatter(t_loc, cd0, cd1, cd2, row1d, zeros1d, e0, epw):
    sizes = [C4] * (epw // C4)
    if epw % C4:
        sizes.append(epw % C4)
    mesh = plsc.VectorSubcoreMesh(core_axis_name="c", subcore_axis_name="s")
    sl = ACC // NS

    @functools.partial(
        pl.kernel,
        out_type=jax.ShapeDtypeStruct((NC, ACC // 128, 128), jnp.float32),
        mesh=mesh,
        scratch_types=[
            pltpu.VMEM((C4,), jnp.int32),
            pltpu.VMEM((C4,), jnp.float32),
            pltpu.VMEM((C4,), jnp.float32),
            pltpu.VMEM((C4,), jnp.float32),
            pltpu.VMEM((C4,), jnp.float32),
            pltpu.VMEM((ACC,), jnp.float32),
            pltpu.VMEM((NS, sl), jnp.float32),
            pltpu.VMEM((sl // 128, 128), jnp.float32),
            pltpu.VMEM_SHARED((NS, ACC), jnp.float32),
        ],
        compiler_params=pltpu.CompilerParams(needs_layout_passes=False),
    )
    def k(t_hbm, cd0_hbm, cd1_hbm, cd2_hbm, row_hbm, zero_hbm, out_hbm,
          rowv, tv, c0v, c1v, c2v, accl, buf2, res, stage):
        cid = lax.axis_index("c")
        sid = lax.axis_index("s")
        wid = cid * NS + sid

        pltpu.sync_copy(zero_hbm, accl)

        loff = 0
        for sz in sizes:
            lbase = wid * epw + loff
            pltpu.sync_copy(row_hbm.at[pl.ds(e0 + lbase, sz)],
                            rowv.at[pl.ds(0, sz)])
            pltpu.sync_copy(t_hbm.at[pl.ds(lbase, sz)], tv.at[pl.ds(0, sz)])
            pltpu.sync_copy(cd0_hbm.at[pl.ds(e0 + lbase, sz)],
                            c0v.at[pl.ds(0, sz)])
            pltpu.sync_copy(cd1_hbm.at[pl.ds(e0 + lbase, sz)],
                            c1v.at[pl.ds(0, sz)])
            pltpu.sync_copy(cd2_hbm.at[pl.ds(e0 + lbase, sz)],
                            c2v.at[pl.ds(0, sz)])

            def grp(g, c2):
                rv = rowv[pl.ds(g * 16, 16)]
                th = tv[pl.ds(g * 16, 16)]
                f0 = rv * 3
                for d, cdv in enumerate((c0v, c1v, c2v)):
                    cd_d = cdv[pl.ds(g * 16, 16)]
                    plsc.addupdate_scatter(accl, [f0 + d], cd_d * th)
                return c2

            lax.fori_loop(0, sz // 16, grp, 0)
            loff += sz

        pltpu.sync_copy(accl, stage.at[sid])
        plsc.subcore_barrier()
        pltpu.sync_copy(stage.at[:, pl.ds(sid * sl, sl)], buf2)

        def colsum_rows(jr, carry):
            for jc in range(8):
                j = jr * 8 + jc
                acc16 = buf2[0, pl.ds(j * 16, 16)]
                for r in range(1, NS):
                    acc16 = acc16 + buf2[r, pl.ds(j * 16, 16)]
                res[jr, pl.ds(jc * 16, 16)] = acc16
            return carry

        lax.fori_loop(0, sl // 128, colsum_rows, 0)
        pltpu.sync_copy(res, out_hbm.at[cid, pl.ds(sid * (sl // 128),
                                                   sl // 128)])

    return k(t_loc, cd0, cd1, cd2, row1d, zeros1d)


def kernel(h, coord, edge_index, coord_diff, edge_attr, W1, b1, W2, b2, W3):
    n, hdim = h.shape
    e = edge_index.shape[1]

    w1a = W1[:, :hdim].astype(jnp.bfloat16)
    w1b = W1[:, hdim:2 * hdim].astype(jnp.bfloat16)
    w1c = W1[:, 2 * hdim:].astype(jnp.bfloat16)

    a_t, b_t = _node_ab(h, w1a, w1b)

    row = edge_index[0]
    col = edge_index[1]

    h0 = 204800
    gpad_a = 204800
    gpad_b = 122880
    epw_a = h0 // NW
    epw_b = (e - h0) // NW

    ea_t = jnp.pad(edge_attr, ((0, EPAD - e), (0, 0))).T
    b1r = b1.reshape(1, -1)
    b2r = b2.reshape(1, -1)
    w2c = W2.astype(jnp.bfloat16)
    w3c = W3.astype(jnp.bfloat16)
    zeros1d = jnp.zeros((ACC,), dtype=jnp.float32)
    cd0, cd1, cd2 = coord_diff[:, 0], coord_diff[:, 1], coord_diff[:, 2]

    g_a = _gather_sum(a_t, b_t, row, col, 0, epw_a, gpad_a)
    g_b = _gather_sum(a_t, b_t, row, col, h0, epw_b, gpad_b)

    t_a = _edge_mlp(g_a, ea_t, 0, w1c, b1r, w2c, b2r, w3c)
    t_b = _edge_mlp(g_b, ea_t, h0 // BE, w1c, b1r, w2c, b2r, w3c)

    pa = _segment_scatter(t_a, cd0, cd1, cd2, row, zeros1d, 0, epw_a)
    pb = _segment_scatter(t_b, cd0, cd1, cd2, row, zeros1d, h0, epw_b)

    agg = ((pa[0] + pa[1]) + (pb[0] + pb[1])).reshape(-1)[:3 * n].reshape(n, 3)
    return coord + agg
